# baseline (speedup 1.0000x reference)
import jax
import jax.numpy as jnp
from jax import lax
from jax.experimental import pallas as pl
from jax.experimental.pallas import tpu as pltpu
from jax.experimental.pallas import tpu_sc as plsc

_N_EMBD = 768
_N_TOKEN = 77
_TOK_PAD = 80
_BATCH = 1024
_LANES = 16
_NUM_CORES = 2
_NUM_SUBCORES = 16
_NW = _NUM_CORES * _NUM_SUBCORES
_ROWS_PER_W = _BATCH // _NW
_NBUF = 5
_CHUNK = 16


def _embed_body(tok_hbm, table_hbm, pos_hbm, pid_hbm, out_hbm,
                b0, b1, b2, b3, b4, p0, p1, p2, p3, p4,
                x0, x1, x2, x3, x4,
                g0, g1, g2, g3, g4, w0, w1, w2, w3, w4,
                s0, s1, s2, s3, s4):
    bufs = (b0, b1, b2, b3, b4)
    pbufs = (p0, p1, p2, p3, p4)
    ibufs = (x0, x1, x2, x3, x4)
    gsem = (g0, g1, g2, g3, g4)
    wsem = (w0, w1, w2, w3, w4)
    isem = (s0, s1, s2, s3, s4)

    wid = lax.axis_index("subcore") * _NUM_CORES + lax.axis_index("core")
    base_row = wid * _ROWS_PER_W

    for j in range(_NBUF):
        pltpu.async_copy(pid_hbm.at[pl.ds(j * _CHUNK, _CHUNK)],
                         ibufs[j], isem[j])
    for j in range(_NBUF):
        pltpu.make_async_copy(pid_hbm.at[pl.ds(j * _CHUNK, _CHUNK)],
                              ibufs[j], isem[j]).wait()
        pltpu.async_copy(pos_hbm.at[ibufs[j]], pbufs[j], gsem[j])
    for j in range(_NBUF):
        pltpu.make_async_copy(pos_hbm.at[ibufs[j]], pbufs[j],
                              gsem[j]).wait()

    def i_pair(row, j):
        src = tok_hbm.at[base_row + row, pl.ds(j * _CHUNK, _CHUNK)]
        return src, ibufs[j]

    def i_start(row, j):
        src, dst = i_pair(row, j)
        pltpu.async_copy(src, dst, isem[j])

    def i_wait(row, j):
        src, dst = i_pair(row, j)
        pltpu.make_async_copy(src, dst, isem[j]).wait()

    def g_start(row, j):
        pltpu.async_copy(table_hbm.at[ibufs[j]], bufs[j], gsem[j])

    def g_wait(row, j):
        pltpu.make_async_copy(
            table_hbm.at[ibufs[j]], bufs[j], gsem[j]).wait()

    def w_pair(row, j):
        return (bufs[j],
                out_hbm.at[base_row + row, pl.ds(j * _CHUNK, _CHUNK)])

    def w_start(row, j):
        src, dst = w_pair(row, j)
        pltpu.async_copy(src, dst, wsem[j])

    def w_wait(row, j):
        src, dst = w_pair(row, j)
        pltpu.make_async_copy(src, dst, wsem[j]).wait()

    for j in range(_NBUF):
        i_start(0, j)
    for j in (0, 1):
        i_wait(0, j)
        g_start(0, j)

    @pl.loop(0, _ROWS_PER_W)
    def _(i):
        for j in range(_NBUF):
            g_wait(i, j)

            @pl.when(i <= _ROWS_PER_W - 2)
            def _():
                i_start(i + 1, j)

            if j < 3:
                jp = j + 2

                @pl.when(i >= 1)
                def _():
                    w_wait(i - 1, jp)

                i_wait(i, jp)
                g_start(i, jp)
            else:
                jp = j - 3
                w_wait(i, jp)

                @pl.when(i <= _ROWS_PER_W - 2)
                def _():
                    i_wait(i + 1, jp)
                    g_start(i + 1, jp)

            @pl.loop(0, _CHUNK)
            def _(r):
                for c in range(0, _N_EMBD, _LANES):
                    sl = pl.ds(c, _LANES)
                    plsc.addupdate(bufs[j].at[r, sl], pbufs[j][r, sl])

            w_start(i, j)

    for j in (2, 3, 4):
        w_wait(_ROWS_PER_W - 1, j)


@jax.jit
def _embed(tokens, token_embedding, position_embedding):
    tok_pad = jnp.pad(tokens, ((0, 0), (0, _TOK_PAD - _N_TOKEN)))
    pos_ids = jnp.pad(jnp.arange(_N_TOKEN, dtype=jnp.int32),
                      (0, _TOK_PAD - _N_TOKEN))
    mesh = plsc.VectorSubcoreMesh(
        core_axis_name="core", subcore_axis_name="subcore")
    kern = pl.kernel(
        _embed_body,
        out_type=jax.ShapeDtypeStruct((_BATCH, _TOK_PAD, _N_EMBD),
                                      jnp.float32),
        mesh=mesh,
        scratch_types=(
            [pltpu.VMEM((_CHUNK, _N_EMBD), jnp.float32)
             for _ in range(2 * _NBUF)]
            + [pltpu.VMEM((_CHUNK,), jnp.int32) for _ in range(_NBUF)]
            + [pltpu.SemaphoreType.DMA for _ in range(3 * _NBUF)]
        ),
    )
    out = kern(tok_pad, token_embedding, position_embedding, pos_ids)
    return out[:, :_N_TOKEN, :]


def kernel(tokens, token_embedding, position_embedding):
    return _embed(tokens, token_embedding, position_embedding)


# speedup vs baseline: 1.5336x; 1.5336x over previous
import jax
import jax.numpy as jnp
from jax import lax
from jax.experimental import pallas as pl
from jax.experimental.pallas import tpu as pltpu
from jax.experimental.pallas import tpu_sc as plsc

_N_EMBD = 768
_N_TOKEN = 77
_BATCH = 1024
_LANES = 16
_NUM_CORES = 2
_NUM_SUBCORES = 16
_NW = _NUM_CORES * _NUM_SUBCORES
_ROWS_PER_W = _BATCH // _NW
_NBUF = 6
_COL_OFF = (0, 16, 32, 48, 64, 72)
_COL_ROWS = (16, 16, 16, 16, 8, 5)


def _embed_body(tok_hbm, table_hbm, pos_hbm, pid_hbm, out_hbm,
                b0, b1, b2, b3, b4, b5, p0, p1, p2, p3, p4, p5,
                x0, x1, x2, x3, x4, x5,
                g0, g1, g2, g3, g4, g5, w0, w1, w2, w3, w4, w5,
                s0, s1, s2, s3, s4, s5):
    bufs = (b0, b1, b2, b3, b4, b5)
    pbufs = (p0, p1, p2, p3, p4, p5)
    ibufs = (x0, x1, x2, x3, x4, x5)
    gsem = (g0, g1, g2, g3, g4, g5)
    wsem = (w0, w1, w2, w3, w4, w5)
    isem = (s0, s1, s2, s3, s4, s5)

    wid = lax.axis_index("subcore") * _NUM_CORES + lax.axis_index("core")
    base_row = wid * _ROWS_PER_W

    for j in range(_NBUF):
        pltpu.async_copy(pid_hbm.at[pl.ds(_COL_OFF[j], _COL_ROWS[j])],
                         ibufs[j], isem[j])
    for j in range(_NBUF):
        pltpu.make_async_copy(pid_hbm.at[pl.ds(_COL_OFF[j], _COL_ROWS[j])],
                              ibufs[j], isem[j]).wait()
        pltpu.async_copy(pos_hbm.at[ibufs[j]], pbufs[j], gsem[j])
    for j in range(_NBUF):
        pltpu.make_async_copy(pos_hbm.at[ibufs[j]], pbufs[j],
                              gsem[j]).wait()

    def i_pair(row, j):
        src = tok_hbm.at[base_row + row,
                         pl.ds(_COL_OFF[j], _COL_ROWS[j])]
        return src, ibufs[j]

    def i_start(row, j):
        src, dst = i_pair(row, j)
        pltpu.async_copy(src, dst, isem[j])

    def i_wait(row, j):
        src, dst = i_pair(row, j)
        pltpu.make_async_copy(src, dst, isem[j]).wait()

    def g_start(row, j):
        pltpu.async_copy(table_hbm.at[ibufs[j]], bufs[j], gsem[j])

    def g_wait(row, j):
        pltpu.make_async_copy(
            table_hbm.at[ibufs[j]], bufs[j], gsem[j]).wait()

    def w_pair(row, j):
        return (bufs[j],
                out_hbm.at[base_row + row, pl.ds(_COL_OFF[j], _COL_ROWS[j])])

    def w_start(row, j):
        src, dst = w_pair(row, j)
        pltpu.async_copy(src, dst, wsem[j])

    def w_wait(row, j):
        src, dst = w_pair(row, j)
        pltpu.make_async_copy(src, dst, wsem[j]).wait()

    for j in range(_NBUF):
        i_start(0, j)
    for j in (0, 1):
        i_wait(0, j)
        g_start(0, j)

    @pl.loop(0, _ROWS_PER_W)
    def _(i):
        for j in range(_NBUF):
            rows = _COL_ROWS[j]
            g_wait(i, j)

            @pl.when(i <= _ROWS_PER_W - 2)
            def _():
                i_start(i + 1, j)

            if j < 4:
                jp = j + 2

                @pl.when(i >= 1)
                def _():
                    w_wait(i - 1, jp)

                i_wait(i, jp)
                g_start(i, jp)
            else:
                jp = j - 4
                w_wait(i, jp)

                @pl.when(i <= _ROWS_PER_W - 2)
                def _():
                    i_wait(i + 1, jp)
                    g_start(i + 1, jp)

            @pl.loop(0, rows)
            def _(r):
                for c in range(0, _N_EMBD, _LANES):
                    sl = pl.ds(c, _LANES)
                    plsc.addupdate(bufs[j].at[r, sl], pbufs[j][r, sl])

            w_start(i, j)

    for j in (2, 3, 4, 5):
        w_wait(_ROWS_PER_W - 1, j)


@jax.jit
def _embed(tokens, token_embedding, position_embedding):
    pos_ids = jnp.arange(_N_TOKEN, dtype=jnp.int32)
    mesh = plsc.VectorSubcoreMesh(
        core_axis_name="core", subcore_axis_name="subcore")
    kern = pl.kernel(
        _embed_body,
        out_type=jax.ShapeDtypeStruct((_BATCH, _N_TOKEN, _N_EMBD),
                                      jnp.float32),
        mesh=mesh,
        scratch_types=(
            [pltpu.VMEM((_COL_ROWS[j], _N_EMBD), jnp.float32)
             for j in range(_NBUF)] * 2
            + [pltpu.VMEM((_COL_ROWS[j],), jnp.int32) for j in range(_NBUF)]
            + [pltpu.SemaphoreType.DMA for _ in range(3 * _NBUF)]
        ),
    )
    return kern(tokens, token_embedding, position_embedding, pos_ids)


def kernel(tokens, token_embedding, position_embedding):
    return _embed(tokens, token_embedding, position_embedding)


# speedup vs baseline: 1.6025x; 1.0449x over previous
import jax
import jax.numpy as jnp
from jax import lax
from jax.experimental import pallas as pl
from jax.experimental.pallas import tpu as pltpu
from jax.experimental.pallas import tpu_sc as plsc

_N_EMBD = 768
_N_TOKEN = 77
_BATCH = 1024
_LANES = 16
_PANEL = 128
_NPANEL = _N_EMBD // _PANEL
_NUM_CORES = 2
_NUM_SUBCORES = 16
_NW = _NUM_CORES * _NUM_SUBCORES
_ROWS_PER_W = _BATCH // _NW


def _embed_body(tok_hbm, table_hbm, pos_hbm, out_hbm,
                b0, b1, b2, b3, b4, b5, p0, p1, p2, p3, p4, p5,
                x0, x1,
                g0, g1, g2, g3, g4, g5, w0, w1, w2, w3, w4, w5,
                s0, s1):
    bufs = (b0, b1, b2, b3, b4, b5)
    pbufs = (p0, p1, p2, p3, p4, p5)
    ibufs = (x0, x1)
    gsem = (g0, g1, g2, g3, g4, g5)
    wsem = (w0, w1, w2, w3, w4, w5)
    isem = (s0, s1)

    wid = lax.axis_index("subcore") * _NUM_CORES + lax.axis_index("core")
    base_row = wid * _ROWS_PER_W

    for cb in range(_NPANEL):
        pltpu.sync_copy(pos_hbm.at[:, pl.ds(cb * _PANEL, _PANEL)], pbufs[cb])

    def i_pair(row, par):
        return tok_hbm.at[base_row + row], ibufs[par]

    def i_start(row, par):
        src, dst = i_pair(row, par)
        pltpu.async_copy(src, dst, isem[par])

    def i_wait(row, par):
        src, dst = i_pair(row, par)
        pltpu.make_async_copy(src, dst, isem[par]).wait()

    def g_start(row, par, cb):
        pltpu.async_copy(
            table_hbm.at[ibufs[par], pl.ds(cb * _PANEL, _PANEL)],
            bufs[cb], gsem[cb])

    def g_wait(row, par, cb):
        pltpu.make_async_copy(
            table_hbm.at[ibufs[par], pl.ds(cb * _PANEL, _PANEL)],
            bufs[cb], gsem[cb]).wait()

    def w_pair(row, cb):
        return (bufs[cb],
                out_hbm.at[base_row + row, :, pl.ds(cb * _PANEL, _PANEL)])

    def w_start(row, cb):
        src, dst = w_pair(row, cb)
        pltpu.async_copy(src, dst, wsem[cb])

    def w_wait(row, cb):
        src, dst = w_pair(row, cb)
        pltpu.make_async_copy(src, dst, wsem[cb]).wait()

    i_start(0, 0)
    i_wait(0, 0)
    g_start(0, 0, 0)
    g_start(0, 0, 1)

    @pl.loop(0, _ROWS_PER_W // 2)
    def _(i2):
        for h in (0, 1):
            row = 2 * i2 + h
            for cb in range(_NPANEL):
                g_wait(row, h, cb)

                if cb == 0:
                    if h == 0:
                        i_start(row + 1, 1)
                    else:
                        @pl.when(i2 <= _ROWS_PER_W // 2 - 2)
                        def _():
                            i_start(row + 1, 0)

                if cb < 4:
                    jp = cb + 2
                    if h == 0:
                        @pl.when(i2 >= 1)
                        def _():
                            w_wait(row - 1, jp)
                    else:
                        w_wait(row - 1, jp)
                    g_start(row, h, jp)
                else:
                    jp = cb - 4
                    w_wait(row, jp)
                    if cb == 4:
                        if h == 0:
                            i_wait(row + 1, 1)
                            g_start(row + 1, 1, jp)
                        else:
                            @pl.when(i2 <= _ROWS_PER_W // 2 - 2)
                            def _():
                                i_wait(row + 1, 0)
                                g_start(row + 1, 0, jp)
                    else:
                        if h == 0:
                            g_start(row + 1, 1, jp)
                        else:
                            @pl.when(i2 <= _ROWS_PER_W // 2 - 2)
                            def _():
                                g_start(row + 1, 0, jp)

                @pl.loop(0, _N_TOKEN)
                def _(r):
                    for c in range(0, _PANEL, _LANES):
                        sl = pl.ds(c, _LANES)
                        plsc.addupdate(bufs[cb].at[r, sl], pbufs[cb][r, sl])

                w_start(row, cb)

    for cb in (2, 3, 4, 5):
        w_wait(_ROWS_PER_W - 1, cb)


@jax.jit
def _embed(tokens, token_embedding, position_embedding):
    mesh = plsc.VectorSubcoreMesh(
        core_axis_name="core", subcore_axis_name="subcore")
    kern = pl.kernel(
        _embed_body,
        out_type=jax.ShapeDtypeStruct((_BATCH, _N_TOKEN, _N_EMBD),
                                      jnp.float32),
        mesh=mesh,
        scratch_types=(
            [pltpu.VMEM((_N_TOKEN, _PANEL), jnp.float32)
             for _ in range(2 * _NPANEL)]
            + [pltpu.VMEM((_N_TOKEN,), jnp.int32) for _ in range(2)]
            + [pltpu.SemaphoreType.DMA for _ in range(2 * _NPANEL + 2)]
        ),
    )
    return kern(tokens, token_embedding, position_embedding)


def kernel(tokens, token_embedding, position_embedding):
    return _embed(tokens, token_embedding, position_embedding)


# speedup vs baseline: 2.1923x; 1.3681x over previous
import jax
import jax.numpy as jnp
from jax import lax
from jax.experimental import pallas as pl
from jax.experimental.pallas import tpu as pltpu
from jax.experimental.pallas import tpu_sc as plsc

_N_EMBD = 768
_N_TOKEN = 77
_BATCH = 1024
_LANES = 16
_PANEL = 128
_NPANEL = _N_EMBD // _PANEL
_BLK = 32
_NBLK = _BATCH // _BLK
_NUM_CORES = 2
_NUM_SUBCORES = 16
_NW = _NUM_CORES * _NUM_SUBCORES
_PAIRS_PER_W = _N_TOKEN * _NBLK // _NW


def _embed_body(tok_hbm, table_hbm, pos_hbm, out_hbm,
                pos_v, idx_v, b0, b1, b2, b3, b4, b5,
                g0, g1, g2, g3, g4, g5, w0, w1, w2, w3, w4, w5, si):
    bufs = (b0, b1, b2, b3, b4, b5)
    gsem = (g0, g1, g2, g3, g4, g5)
    wsem = (w0, w1, w2, w3, w4, w5)

    wid = lax.axis_index("subcore") * _NUM_CORES + lax.axis_index("core")
    base_pair = wid * _PAIRS_PER_W

    pltpu.sync_copy(pos_hbm, pos_v)
    pltpu.async_copy(
        tok_hbm.at[pl.ds(base_pair * _BLK, _PAIRS_PER_W * _BLK)],
        idx_v, si).wait()

    def g_start(q, cb):
        pltpu.async_copy(
            table_hbm.at[idx_v.at[pl.ds(q * _BLK, _BLK)],
                         pl.ds(cb * _PANEL, _PANEL)],
            bufs[cb], gsem[cb])

    def g_wait(q, cb):
        pltpu.make_async_copy(
            table_hbm.at[idx_v.at[pl.ds(q * _BLK, _BLK)],
                         pl.ds(cb * _PANEL, _PANEL)],
            bufs[cb], gsem[cb]).wait()

    def w_pair(q, cb):
        pair = base_pair + q
        t = pair // _NBLK
        bb = lax.rem(pair, _NBLK)
        return (bufs[cb],
                out_hbm.at[t, pl.ds(bb * _BLK, _BLK),
                           pl.ds(cb * _PANEL, _PANEL)])

    def w_start(q, cb):
        src, dst = w_pair(q, cb)
        pltpu.async_copy(src, dst, wsem[cb])

    def w_wait(q, cb):
        src, dst = w_pair(q, cb)
        pltpu.make_async_copy(src, dst, wsem[cb]).wait()

    g_start(0, 0)
    g_start(0, 1)

    @pl.loop(0, _PAIRS_PER_W)
    def _(q):
        t = (base_pair + q) // _NBLK

        for cb in range(_NPANEL):
            g_wait(q, cb)

            if cb < 4:
                jp = cb + 2

                @pl.when(q >= 1)
                def _():
                    w_wait(q - 1, jp)

                g_start(q, jp)
            else:
                jp = cb - 4
                w_wait(q, jp)

                @pl.when(q <= _PAIRS_PER_W - 2)
                def _():
                    g_start(q + 1, jp)

            vals = [pos_v[t, pl.ds(cb * _PANEL + c, _LANES)]
                    for c in range(0, _PANEL, _LANES)]

            @pl.loop(0, _BLK)
            def _(r):
                for k, c in enumerate(range(0, _PANEL, _LANES)):
                    plsc.addupdate(bufs[cb].at[r, pl.ds(c, _LANES)], vals[k])

            w_start(q, cb)

    for cb in (2, 3, 4, 5):
        w_wait(_PAIRS_PER_W - 1, cb)


@jax.jit
def _embed(tokens, token_embedding, position_embedding):
    tok_flat = tokens.T.reshape(_N_TOKEN * _BATCH)
    mesh = plsc.VectorSubcoreMesh(
        core_axis_name="core", subcore_axis_name="subcore")
    kern = pl.kernel(
        _embed_body,
        out_type=jax.ShapeDtypeStruct((_N_TOKEN, _BATCH, _N_EMBD),
                                      jnp.float32),
        mesh=mesh,
        scratch_types=(
            [pltpu.VMEM((_N_TOKEN, _N_EMBD), jnp.float32),
             pltpu.VMEM((_PAIRS_PER_W * _BLK,), jnp.int32)]
            + [pltpu.VMEM((_BLK, _PANEL), jnp.float32)
               for _ in range(_NPANEL)]
            + [pltpu.SemaphoreType.DMA for _ in range(2 * _NPANEL + 1)]
        ),
    )
    out_t = kern(tok_flat, token_embedding, position_embedding)
    return jnp.transpose(out_t, (1, 0, 2))


def kernel(tokens, token_embedding, position_embedding):
    return _embed(tokens, token_embedding, position_embedding)


# speedup vs baseline: 2.7280x; 1.2444x over previous
import jax
import jax.numpy as jnp
from jax import lax
from jax.experimental import pallas as pl
from jax.experimental.pallas import tpu as pltpu
from jax.experimental.pallas import tpu_sc as plsc

_N_EMBD = 768
_N_TOKEN = 77
_BATCH = 1024
_LANES = 16
_PANEL = 128
_NPANEL = _N_EMBD // _PANEL
_BLK = 32
_NBLK = _BATCH // _BLK
_NUM_CORES = 2
_NUM_SUBCORES = 16
_NW = _NUM_CORES * _NUM_SUBCORES
_PAIRS_PER_W = _N_TOKEN * _NBLK // _NW


def _embed_body(tok_hbm, table_hbm, pos_hbm, out_hbm,
                pos_v, idx_v, b0, b1, b2, b3, b4, b5,
                g0, g1, g2, g3, g4, g5, w0, w1, w2, w3, w4, w5, si):
    bufs = (b0, b1, b2, b3, b4, b5)
    gsem = (g0, g1, g2, g3, g4, g5)
    wsem = (w0, w1, w2, w3, w4, w5)

    wid = lax.axis_index("subcore") * _NUM_CORES + lax.axis_index("core")
    base_pair = wid * _PAIRS_PER_W

    pltpu.sync_copy(pos_hbm, pos_v)
    pltpu.async_copy(
        tok_hbm.at[pl.ds(base_pair * _BLK, _PAIRS_PER_W * _BLK)],
        idx_v, si).wait()

    def g_start(q, cb):
        pltpu.async_copy(
            table_hbm.at[idx_v.at[pl.ds(q * _BLK, _BLK)],
                         pl.ds(cb * _PANEL, _PANEL)],
            bufs[cb], gsem[cb])

    def g_wait(q, cb):
        pltpu.make_async_copy(
            table_hbm.at[idx_v.at[pl.ds(q * _BLK, _BLK)],
                         pl.ds(cb * _PANEL, _PANEL)],
            bufs[cb], gsem[cb]).wait()

    def w_pair(q, cb):
        pair = base_pair + q
        t = pair // _NBLK
        bb = lax.rem(pair, _NBLK)
        return (bufs[cb],
                out_hbm.at[t, pl.ds(bb * _BLK, _BLK),
                           pl.ds(cb * _PANEL, _PANEL)])

    def w_start(q, cb):
        src, dst = w_pair(q, cb)
        pltpu.async_copy(src, dst, wsem[cb])

    def w_wait(q, cb):
        src, dst = w_pair(q, cb)
        pltpu.make_async_copy(src, dst, wsem[cb]).wait()

    g_start(0, 0)
    g_start(0, 1)
    g_start(0, 2)

    @pl.loop(0, _PAIRS_PER_W)
    def _(q):
        t = (base_pair + q) // _NBLK

        for cb in range(_NPANEL):
            g_wait(q, cb)

            if cb < 3:
                jp = cb + 3

                @pl.when(q >= 1)
                def _():
                    w_wait(q - 1, jp)

                g_start(q, jp)
            else:
                jp = cb - 3
                w_wait(q, jp)

                @pl.when(q <= _PAIRS_PER_W - 2)
                def _():
                    g_start(q + 1, jp)

            vals = [pos_v[t, pl.ds(cb * _PANEL + c, _LANES)]
                    for c in range(0, _PANEL, _LANES)]

            @pl.loop(0, _BLK)
            def _(r):
                for k, c in enumerate(range(0, _PANEL, _LANES)):
                    plsc.addupdate(bufs[cb].at[r, pl.ds(c, _LANES)], vals[k])

            w_start(q, cb)

    for cb in (3, 4, 5):
        w_wait(_PAIRS_PER_W - 1, cb)


@jax.jit
def _embed(tokens, token_embedding, position_embedding):
    tok_flat = tokens.T.reshape(_N_TOKEN * _BATCH)
    mesh = plsc.VectorSubcoreMesh(
        core_axis_name="core", subcore_axis_name="subcore")
    kern = pl.kernel(
        _embed_body,
        out_type=jax.ShapeDtypeStruct((_N_TOKEN, _BATCH, _N_EMBD),
                                      jnp.float32),
        mesh=mesh,
        scratch_types=(
            [pltpu.VMEM((_N_TOKEN, _N_EMBD), jnp.float32),
             pltpu.VMEM((_PAIRS_PER_W * _BLK,), jnp.int32)]
            + [pltpu.VMEM((_BLK, _PANEL), jnp.float32)
               for _ in range(_NPANEL)]
            + [pltpu.SemaphoreType.DMA for _ in range(2 * _NPANEL + 1)]
        ),
    )
    out_t = kern(tok_flat, token_embedding, position_embedding)
    return jnp.transpose(out_t, (1, 0, 2))


def kernel(tokens, token_embedding, position_embedding):
    return _embed(tokens, token_embedding, position_embedding)


# speedup vs baseline: 2.9825x; 1.0933x over previous
import jax
import jax.numpy as jnp
from jax import lax
from jax.experimental import pallas as pl
from jax.experimental.pallas import tpu as pltpu
from jax.experimental.pallas import tpu_sc as plsc

_N_EMBD = 768
_N_TOKEN = 77
_BATCH = 1024
_LANES = 16
_PANEL = 128
_NPANEL = _N_EMBD // _PANEL
_BLK = 32
_NBLK = _BATCH // _BLK
_NUM_CORES = 2
_NUM_SUBCORES = 16
_NW = _NUM_CORES * _NUM_SUBCORES
_PAIRS_PER_W = _N_TOKEN * _NBLK // _NW


def _embed_body(tok_hbm, table_hbm, pos_hbm, out_hbm,
                pos_v, idx_v, b0, b1, b2, b3, b4, b5,
                g0, g1, g2, g3, g4, g5, w0, w1, w2, w3, w4, w5, si):
    bufs = (b0, b1, b2, b3, b4, b5)
    gsem = (g0, g1, g2, g3, g4, g5)
    wsem = (w0, w1, w2, w3, w4, w5)

    wid = lax.axis_index("subcore") * _NUM_CORES + lax.axis_index("core")
    base_pair = wid * _PAIRS_PER_W

    pltpu.sync_copy(pos_hbm, pos_v)
    pltpu.async_copy(
        tok_hbm.at[pl.ds(base_pair * _BLK, _PAIRS_PER_W * _BLK)],
        idx_v, si).wait()

    def g_start(q, cb):
        pltpu.async_copy(
            table_hbm.at[idx_v.at[pl.ds(q * _BLK, _BLK)],
                         pl.ds(cb * _PANEL, _PANEL)],
            bufs[cb], gsem[cb])

    def g_wait(q, cb):
        pltpu.make_async_copy(
            table_hbm.at[idx_v.at[pl.ds(q * _BLK, _BLK)],
                         pl.ds(cb * _PANEL, _PANEL)],
            bufs[cb], gsem[cb]).wait()

    def w_pair(q, cb):
        pair = base_pair + q
        t = pair // _NBLK
        bb = lax.rem(pair, _NBLK)
        return (bufs[cb],
                out_hbm.at[t, pl.ds(bb * _BLK, _BLK),
                           pl.ds(cb * _PANEL, _PANEL)])

    def w_start(q, cb):
        src, dst = w_pair(q, cb)
        pltpu.async_copy(src, dst, wsem[cb])

    def w_wait(q, cb):
        src, dst = w_pair(q, cb)
        pltpu.make_async_copy(src, dst, wsem[cb]).wait()

    for cb in (0, 1, 2, 3):
        g_start(0, cb)

    @pl.loop(0, _PAIRS_PER_W)
    def _(q):
        t = (base_pair + q) // _NBLK

        for cb in range(_NPANEL):
            g_wait(q, cb)

            if cb < 2:
                jp = cb + 4

                @pl.when(q >= 1)
                def _():
                    w_wait(q - 1, jp)

                g_start(q, jp)
            else:
                jp = cb - 2
                w_wait(q, jp)

                @pl.when(q <= _PAIRS_PER_W - 2)
                def _():
                    g_start(q + 1, jp)

            vals = [pos_v[t, pl.ds(cb * _PANEL + c, _LANES)]
                    for c in range(0, _PANEL, _LANES)]

            @pl.loop(0, _BLK)
            def _(r):
                for k, c in enumerate(range(0, _PANEL, _LANES)):
                    plsc.addupdate(bufs[cb].at[r, pl.ds(c, _LANES)], vals[k])

            w_start(q, cb)

    for cb in (4, 5):
        w_wait(_PAIRS_PER_W - 1, cb)


@jax.jit
def _embed(tokens, token_embedding, position_embedding):
    tok_flat = tokens.T.reshape(_N_TOKEN * _BATCH)
    mesh = plsc.VectorSubcoreMesh(
        core_axis_name="core", subcore_axis_name="subcore")
    kern = pl.kernel(
        _embed_body,
        out_type=jax.ShapeDtypeStruct((_N_TOKEN, _BATCH, _N_EMBD),
                                      jnp.float32),
        mesh=mesh,
        scratch_types=(
            [pltpu.VMEM((_N_TOKEN, _N_EMBD), jnp.float32),
             pltpu.VMEM((_PAIRS_PER_W * _BLK,), jnp.int32)]
            + [pltpu.VMEM((_BLK, _PANEL), jnp.float32)
               for _ in range(_NPANEL)]
            + [pltpu.SemaphoreType.DMA for _ in range(2 * _NPANEL + 1)]
        ),
    )
    out_t = kern(tok_flat, token_embedding, position_embedding)
    return jnp.transpose(out_t, (1, 0, 2))


def kernel(tokens, token_embedding, position_embedding):
    return _embed(tokens, token_embedding, position_embedding)
